# baseline (speedup 1.0000x reference)
import functools

import jax
import jax.numpy as jnp
from jax import lax
from jax.experimental import pallas as pl
from jax.experimental.pallas import tpu as pltpu
from jax.experimental.pallas import tpu_sc as plsc

_N = 10000
_D = 128
_E = 320000
_NC = 2
_NS = 16
_CH = 128
_Q = 4
_NBLK = -(-_E // (_CH * _NS * _Q))
_NPW = _NBLK * _Q
_EPAD = _NPW * _CH * _NS
_RT = 640
_NA = _RT * _NS
_HPAD = 8


def _sc_two_sided_accumulate(h_pad, eidx, zero_blk):
    mesh = plsc.VectorSubcoreMesh(core_axis_name="c", subcore_axis_name="s")

    @functools.partial(
        pl.kernel,
        out_type=jax.ShapeDtypeStruct((_NC, _NA, _D), jnp.float32),
        mesh=mesh,
        scratch_types=[
            *[pltpu.VMEM((_Q, _CH), jnp.int32) for _ in range(2)],
            *[pltpu.VMEM((_Q, _CH), jnp.int32) for _ in range(2)],
            *[pltpu.VMEM((_CH, _D), jnp.float32) for _ in range(2)],
            pltpu.VMEM_SHARED((_NA, _D), jnp.float32),
            *[pltpu.SemaphoreType.DMA for _ in range(8)],
        ],
    )
    def k(h_hbm, e_hbm, z_hbm, out_hbm, g0, g1, d0, d1, r0, r1, acc, *sems):
        sg = [g0, g1]
        sd = [d0, d1]
        rows = [r0, r1]
        isg = sems[0:2]
        isd = sems[2:4]
        gsem = sems[4:6]
        ssem = sems[6:8]
        c = lax.axis_index("c")
        s = lax.axis_index("s")

        def idx_start(blk, t):
            pltpu.async_copy(e_hbm.at[c, s, blk], sg[t], isg[t])
            pltpu.async_copy(e_hbm.at[1, s, blk], sd[t], isd[t])

        def idx_wait(blk, t):
            pltpu.make_async_copy(e_hbm.at[c, s, blk], sg[t], isg[t]).wait()
            pltpu.make_async_copy(e_hbm.at[1, s, blk], sd[t], isd[t]).wait()

        def gather_start(t, u, b):
            pltpu.async_copy(h_hbm.at[sg[t].at[u]], rows[b], gsem[b])

        def gather_wait(t, u, b):
            pltpu.make_async_copy(h_hbm.at[sg[t].at[u]], rows[b],
                                  gsem[b]).wait()

        def scatter_start(t, u, b):
            pltpu.async_copy(rows[b], acc.at[sd[t].at[u]], ssem[b], add=True)

        def scatter_wait(t, u, b):
            pltpu.make_async_copy(rows[b], acc.at[sd[t].at[u]],
                                  ssem[b]).wait()

        idx_start(0, 0)
        pltpu.sync_copy(z_hbm, acc.at[pl.ds(s * _RT, _RT)])
        idx_wait(0, 0)
        gather_start(0, 0, 0)
        plsc.subcore_barrier()

        def body(i, carry):
            for p in range(2):
                blk = 2 * i + p
                for u in range(_Q):
                    j = blk * _Q + u
                    b = u % 2
                    gather_wait(p, u, b)
                    scatter_start(p, u, b)

                    @pl.when(j >= 1)
                    def _():
                        scatter_wait(p if u >= 1 else 1 - p,
                                     (u - 1) % _Q, 1 - b)

                    if u == 0:
                        @pl.when(blk + 1 < _NBLK)
                        def _():
                            idx_start(blk + 1, 1 - p)

                    if u + 1 < _Q:
                        gather_start(p, u + 1, 1 - b)
                    else:
                        @pl.when(blk + 1 < _NBLK)
                        def _():
                            idx_wait(blk + 1, 1 - p)
                            gather_start(1 - p, 0, 1 - b)
            return carry

        lax.fori_loop(0, _NBLK // 2, body, 0)
        scatter_wait(1, _Q - 1, (_NPW - 1) % 2)
        plsc.subcore_barrier()

        pltpu.sync_copy(acc.at[pl.ds(s * _RT, _RT)],
                        out_hbm.at[c, pl.ds(s * _RT, _RT)])

    return k(h_pad, eidx, zero_blk)


_BN = 400


def _tc_finalize(partials):
    def body(p_ref, o_ref):
        agg = p_ref[0] - p_ref[1]
        ss = jnp.sum(agg * agg, axis=1, keepdims=True)
        o_ref[...] = agg / (jnp.sqrt(ss) + 1e-7)

    return pl.pallas_call(
        body,
        grid=(_N // _BN,),
        in_specs=[pl.BlockSpec((_NC, _BN, _D), lambda i: (0, i, 0))],
        out_specs=pl.BlockSpec((_BN, _D), lambda i: (i, 0)),
        out_shape=jax.ShapeDtypeStruct((_N, _D), jnp.float32),
    )(partials)


def kernel(h, edge_index, r):
    eidx = jnp.concatenate(
        [edge_index.astype(jnp.int32),
         jnp.full((2, _EPAD - _E), _N, jnp.int32)], axis=1)
    eidx = eidx.reshape(2, _NS, _NBLK, _Q, _CH)
    h_pad = jnp.concatenate(
        [h, jnp.zeros((_HPAD, _D), jnp.float32)], axis=0)
    zero_blk = jnp.zeros((_RT, _D), jnp.float32)
    partials = _sc_two_sided_accumulate(h_pad, eidx, zero_blk)
    return _tc_finalize(partials)
